# baseline (speedup 1.0000x reference)
import jax
import jax.numpy as jnp
from jax import lax
from jax.experimental import pallas as pl
from jax.experimental.pallas import tpu as pltpu

_B = 128
_K = 1024
_DIM = 50257
_KT = 64
_NT = _K // _KT
_DC = 4096
_NS = (_DIM + _DC - 1) // _DC


def _logq_body(q_ref, out_ref):
    out_ref[...] = jnp.log(q_ref[...] + 1e-10).astype(jnp.bfloat16)


_logq = pl.pallas_call(
    _logq_body,
    out_shape=jax.ShapeDtypeStruct((_B, _DIM), jnp.bfloat16),
    compiler_params=pltpu.CompilerParams(vmem_limit_bytes=50 * 1024 * 1024),
)


def _main_body(lq_ref, lk_ref, lab_ref, out_ref, scoresT, selfvec):
    i = pl.program_id(0)
    dn = (((1,), (1,)), ((), ()))

    def _step(lkv, lqc, acc, sacc):
        e = jnp.exp(lkv)
        acc = acc + lax.dot_general(e.astype(jnp.bfloat16), lqc, dn,
                                    preferred_element_type=jnp.float32)
        elk = e * lkv
        for t in range(_DC // 128):
            sacc = sacc + elk[:, t * 128:(t + 1) * 128]
        return acc, sacc

    def _full(c, carry):
        acc, sacc = carry
        lo = pl.multiple_of(c * _DC, _DC)
        return _step(lk_ref[0, :, pl.ds(lo, _DC)], lq_ref[:, pl.ds(lo, _DC)],
                     acc, sacc)

    acc = jnp.zeros((_KT, _B), jnp.float32)
    sacc = jnp.zeros((_KT, 128), jnp.float32)
    acc, sacc = lax.fori_loop(0, _NS - 1, _full, (acc, sacc))
    lo = (_NS - 1) * _DC
    lkv = lax.pad(lk_ref[0, :, lo:_DIM], jnp.float32(-1e30),
                  ((0, 0, 0), (0, _DC - (_DIM - lo), 0)))
    lqc = lax.pad(lq_ref[:, lo:_DIM], jnp.bfloat16(0.0),
                  ((0, 0, 0), (0, _DC - (_DIM - lo), 0)))
    acc, sacc = _step(lkv, lqc, acc, sacc)
    self_col = jnp.sum(sacc, axis=1, keepdims=True)
    scoresT[pl.ds(i * _KT, _KT), :] = acc - self_col

    @pl.when(i == _NT - 1)
    def _finish():
        s = scoresT[...]
        iota_k = lax.broadcasted_iota(jnp.int32, (_K, _B), 0)
        labels = lab_ref[...]
        total = jnp.zeros((1, _B), jnp.int32)
        for _ in range(3):
            m = jnp.max(s, axis=0, keepdims=True)
            first = jnp.min(jnp.where(s == m, iota_k, _K), axis=0, keepdims=True)
            sel = iota_k == first
            total += jnp.sum(jnp.where(sel, labels, 0), axis=0, keepdims=True)
            s = jnp.where(sel, -jnp.inf, s)
        out_ref[...] = (total >= 2).astype(jnp.int32)


_knn_vote = pl.pallas_call(
    _main_body,
    grid=(_NT,),
    in_specs=[
        pl.BlockSpec((_B, _DIM), lambda i: (0, 0)),
        pl.BlockSpec((1, _KT, _DIM), lambda i: (i, 0, 0)),
        pl.BlockSpec((_K, 1), lambda i: (0, 0)),
    ],
    out_specs=pl.BlockSpec((1, _B), lambda i: (0, 0)),
    out_shape=jax.ShapeDtypeStruct((1, _B), jnp.int32),
    scratch_shapes=[
        pltpu.VMEM((_K, _B), jnp.float32),
        pltpu.VMEM((_KT, 128), jnp.float32),
    ],
    compiler_params=pltpu.CompilerParams(
        dimension_semantics=("arbitrary",), vmem_limit_bytes=60 * 1024 * 1024),
)


def kernel(query, log_k, labels):
    lq = _logq(query)
    labels2 = labels.astype(jnp.int32).reshape(_K, 1)
    out = _knn_vote(lq, log_k.reshape(_NT, _KT, _DIM), labels2)
    return out.reshape(_B)


# speedup vs baseline: 1.0954x; 1.0954x over previous
import jax
import jax.numpy as jnp
from jax import lax
from jax.experimental import pallas as pl
from jax.experimental.pallas import tpu as pltpu

_B = 128
_K = 1024
_DIM = 50257
_KT = 64
_NT = _K // _KT
_DC = 4096
_NS = (_DIM + _DC - 1) // _DC


def _logq_body(q_ref, out_ref):
    out_ref[...] = jnp.log(q_ref[...] + 1e-10).astype(jnp.bfloat16)


_logq = pl.pallas_call(
    _logq_body,
    out_shape=jax.ShapeDtypeStruct((_B, _DIM), jnp.bfloat16),
    compiler_params=pltpu.CompilerParams(vmem_limit_bytes=50 * 1024 * 1024),
)


def _main_body(lq_ref, lk_ref, lab_ref, out_ref, scoresT, selfvec):
    i = pl.program_id(0)
    dn = (((1,), (1,)), ((), ()))

    def _step(lkv, lqc, acc, sacc):
        e = jnp.exp(lkv)
        acc = acc + lax.dot_general(e.astype(jnp.bfloat16), lqc, dn,
                                    preferred_element_type=jnp.float32)
        elk = e * lkv
        for t in range(_DC // 128):
            sacc = sacc + elk[:, t * 128:(t + 1) * 128]
        return acc, sacc

    acc = jnp.zeros((_KT, _B), jnp.float32)
    sacc = jnp.zeros((_KT, 128), jnp.float32)
    for c in range(_NS - 1):
        lo = c * _DC
        acc, sacc = _step(lk_ref[0, :, lo:lo + _DC], lq_ref[:, lo:lo + _DC],
                          acc, sacc)
    lo = (_NS - 1) * _DC
    lkv = lax.pad(lk_ref[0, :, lo:_DIM], jnp.float32(-1e30),
                  ((0, 0, 0), (0, _DC - (_DIM - lo), 0)))
    lqc = lax.pad(lq_ref[:, lo:_DIM], jnp.bfloat16(0.0),
                  ((0, 0, 0), (0, _DC - (_DIM - lo), 0)))
    acc, sacc = _step(lkv, lqc, acc, sacc)
    self_col = jnp.sum(sacc, axis=1, keepdims=True)
    scoresT[pl.ds(i * _KT, _KT), :] = acc - self_col

    @pl.when(i == _NT - 1)
    def _finish():
        s = scoresT[...]
        iota_k = lax.broadcasted_iota(jnp.int32, (_K, _B), 0)
        labels = lab_ref[...]
        total = jnp.zeros((1, _B), jnp.int32)
        for _ in range(3):
            m = jnp.max(s, axis=0, keepdims=True)
            first = jnp.min(jnp.where(s == m, iota_k, _K), axis=0, keepdims=True)
            sel = iota_k == first
            total += jnp.sum(jnp.where(sel, labels, 0), axis=0, keepdims=True)
            s = jnp.where(sel, -jnp.inf, s)
        out_ref[...] = (total >= 2).astype(jnp.int32)


_knn_vote = pl.pallas_call(
    _main_body,
    grid=(_NT,),
    in_specs=[
        pl.BlockSpec((_B, _DIM), lambda i: (0, 0)),
        pl.BlockSpec((1, _KT, _DIM), lambda i: (i, 0, 0)),
        pl.BlockSpec((_K, 1), lambda i: (0, 0)),
    ],
    out_specs=pl.BlockSpec((1, _B), lambda i: (0, 0)),
    out_shape=jax.ShapeDtypeStruct((1, _B), jnp.int32),
    scratch_shapes=[
        pltpu.VMEM((_K, _B), jnp.float32),
        pltpu.VMEM((_KT, 128), jnp.float32),
    ],
    compiler_params=pltpu.CompilerParams(
        dimension_semantics=("arbitrary",), vmem_limit_bytes=60 * 1024 * 1024),
)


def kernel(query, log_k, labels):
    lq = _logq(query)
    labels2 = labels.astype(jnp.int32).reshape(_K, 1)
    out = _knn_vote(lq, log_k.reshape(_NT, _KT, _DIM), labels2)
    return out.reshape(_B)
